# Initial kernel scaffold; baseline (speedup 1.0000x reference)
import jax
import jax.numpy as jnp
from jax.experimental import pallas as pl


def kernel(x, rel_pos_embedding):
    raise NotImplementedError("write your pallas kernel here")


# speedup vs baseline: 2.8930x; 2.8930x over previous
import functools

import jax
import jax.numpy as jnp
from jax import lax
from jax.experimental import pallas as pl
from jax.experimental.pallas import tpu as pltpu
from jax.experimental.pallas import tpu_sc as plsc

_NC, _NS = 2, 16
_NW = _NC * _NS


@functools.lru_cache(maxsize=None)
def _make_sc_kernel(S, T, D):
    R = S // _NW
    C = S // 2
    WN = C + R - 1
    WP = 16 * ((WN + 15) // 16)

    mesh = plsc.VectorSubcoreMesh(core_axis_name="c", subcore_axis_name="s")

    @functools.partial(
        pl.kernel,
        out_type=jax.ShapeDtypeStruct((S, S, D), jnp.float32),
        mesh=mesh,
        scratch_types=[
            pltpu.VMEM((WP,), jnp.int32),
            pltpu.VMEM((WP, D), jnp.float32),
            pltpu.SemaphoreType.DMA,
            pltpu.SemaphoreType.DMA,
        ],
        compiler_params=pltpu.CompilerParams(use_tc_tiling_on_sc=False),
    )
    def k(table_hbm, out_hbm, idx_v, win_v, gsem, ssem):
        wid = lax.axis_index("s") * _NC + lax.axis_index("c")
        i0 = wid * R
        lanes = lax.iota(jnp.int32, 16)
        for h in range(S // C):
            maxidx = i0 + (R - 1) + (S - 1) - h * C
            for c in range(WP // 16):
                idx_v[pl.ds(c * 16, 16)] = jnp.maximum(
                    maxidx - (c * 16 + lanes), 0)
            gathers = []
            off = 0
            while off < WP:
                n = min(128, WP - off)
                gathers.append(pltpu.async_copy(
                    table_hbm.at[idx_v.at[pl.ds(off, n)]],
                    win_v.at[pl.ds(off, n)], gsem))
                off += n
            for d in gathers:
                d.wait()
            scatters = []
            for r in range(R):
                scatters.append(pltpu.async_copy(
                    win_v.at[pl.ds(R - 1 - r, C)],
                    out_hbm.at[i0 + r, pl.ds(h * C, C)], ssem))
            for d in scatters:
                d.wait()

    return k


def kernel(x, rel_pos_embedding):
    S = x.shape[1]
    T, D = rel_pos_embedding.shape
    return _make_sc_kernel(S, T, D)(rel_pos_embedding)
